# baseline (speedup 1.0000x reference)
import jax
import jax.numpy as jnp
from jax import lax
from jax.experimental import pallas as pl
from jax.experimental.pallas import tpu as pltpu
from jax.experimental.pallas import tpu_sc as plsc

N = 10000
E = 320000
D = 128
NC = 2
NS = 16
NW = NC * NS
CHUNK = 128
NCHT = 80
EPAD = NW * NCHT * CHUNK
NROW = 10240
ROWS_PT = NROW // NS
IH = NCHT // 2
DW = 16

_MESH = plsc.VectorSubcoreMesh(core_axis_name="c", subcore_axis_name="s")
_SC_PARAMS = pltpu.CompilerParams(use_tc_tiling_on_sc=False)


def _deg_body(dst_hbm, out_hbm, idx_d, ones_v, zbuf, acc):
    c = lax.axis_index("c")
    s = lax.axis_index("s")
    wid = c * NS + s
    pltpu.sync_copy(dst_hbm.at[pl.ds(wid * NCHT, NCHT)], idx_d)

    def fill(i, carry):
        ones_v[i, :] = jnp.ones((DW,), jnp.float32)
        zbuf[i, :] = jnp.zeros((DW,), jnp.float32)
        return carry

    lax.fori_loop(0, ROWS_PT, fill, 0)
    pltpu.sync_copy(zbuf, acc.at[pl.ds(s * ROWS_PT, ROWS_PT)])
    plsc.subcore_barrier()

    def edge(j, carry):
        pltpu.sync_copy(ones_v.at[pl.ds(0, CHUNK)], acc.at[idx_d.at[j]], add=True)
        return carry

    lax.fori_loop(0, NCHT, edge, 0)
    plsc.subcore_barrier()
    pltpu.sync_copy(acc.at[pl.ds(s * ROWS_PT, ROWS_PT)],
                    out_hbm.at[c, pl.ds(s * ROWS_PT, ROWS_PT)])


_deg_call = pl.kernel(
    _deg_body,
    out_type=jax.ShapeDtypeStruct((NC, NROW, DW), jnp.float32),
    mesh=_MESH,
    scratch_types=[
        pltpu.VMEM((NCHT, CHUNK), jnp.int32),
        pltpu.VMEM((ROWS_PT, DW), jnp.float32),
        pltpu.VMEM((ROWS_PT, DW), jnp.float32),
        pltpu.VMEM_SHARED((NROW, DW), jnp.float32),
    ],
    compiler_params=_SC_PARAMS,
)


def _scatter_body(g_hbm, src_hbm, dst_hbm, out_hbm, idx_s, idx_d, rb0, rb1, acc,
                  sem0, sem1):
    c = lax.axis_index("c")
    s = lax.axis_index("s")
    wid = c * NS + s

    def zfill(i, carry):
        for j in range(D // 16):
            rb0[i, pl.ds(j * 16, 16)] = jnp.zeros((16,), jnp.float32)
        return carry

    lax.fori_loop(0, CHUNK, zfill, 0)

    def zcopy(k, carry):
        pltpu.sync_copy(rb0, acc.at[pl.ds(s * ROWS_PT + k * CHUNK, CHUNK)])
        return carry

    lax.fori_loop(0, ROWS_PT // CHUNK, zcopy, 0)
    plsc.subcore_barrier()

    for h in range(NCHT // IH):
        pltpu.sync_copy(src_hbm.at[pl.ds(wid * NCHT + h * IH, IH)], idx_s)
        pltpu.sync_copy(dst_hbm.at[pl.ds(wid * NCHT + h * IH, IH)], idx_d)
        pltpu.async_copy(g_hbm.at[idx_s.at[0]], rb0, sem0)

        def pair(i, carry):
            j = 2 * i
            pltpu.async_copy(g_hbm.at[idx_s.at[j + 1]], rb1, sem1)
            pltpu.make_async_copy(g_hbm.at[idx_s.at[j]], rb0, sem0).wait()
            pltpu.sync_copy(rb0, acc.at[idx_d.at[j]], add=True)

            @pl.when(j + 2 < IH)
            def _():
                pltpu.async_copy(g_hbm.at[idx_s.at[j + 2]], rb0, sem0)

            pltpu.make_async_copy(g_hbm.at[idx_s.at[j + 1]], rb1, sem1).wait()
            pltpu.sync_copy(rb1, acc.at[idx_d.at[j + 1]], add=True)
            return carry

        lax.fori_loop(0, IH // 2, pair, 0)
    plsc.subcore_barrier()
    pltpu.sync_copy(acc.at[pl.ds(s * ROWS_PT, ROWS_PT)],
                    out_hbm.at[c, pl.ds(s * ROWS_PT, ROWS_PT)])


_scatter_call = pl.kernel(
    _scatter_body,
    out_type=jax.ShapeDtypeStruct((NC, NROW, D), jnp.float32),
    mesh=_MESH,
    scratch_types=[
        pltpu.VMEM((IH, CHUNK), jnp.int32),
        pltpu.VMEM((IH, CHUNK), jnp.int32),
        pltpu.VMEM((CHUNK, D), jnp.float32),
        pltpu.VMEM((CHUNK, D), jnp.float32),
        pltpu.VMEM_SHARED((NROW, D), jnp.float32),
        pltpu.SemaphoreType.DMA,
        pltpu.SemaphoreType.DMA,
    ],
    compiler_params=_SC_PARAMS,
)

BM = 1000


def _mm_body(x_ref, w_ref, o_ref):
    o_ref[...] = jnp.dot(x_ref[...], w_ref[...], preferred_element_type=jnp.float32)


def _mm(x, w):
    return pl.pallas_call(
        _mm_body,
        grid=(N // BM,),
        in_specs=[pl.BlockSpec((BM, D), lambda i: (i, 0)),
                  pl.BlockSpec((D, D), lambda i: (0, 0))],
        out_specs=pl.BlockSpec((BM, D), lambda i: (i, 0)),
        out_shape=jax.ShapeDtypeStruct((N, D), jnp.float32),
    )(x, w)


def _scale_body(degp_ref, h_ref, dinv_ref, g_ref):
    deg = degp_ref[0, :, 0:1] + degp_ref[1, :, 0:1] + 1.0
    dinv = lax.rsqrt(deg)
    dinv_ref[...] = dinv
    g_ref[...] = h_ref[...] * dinv


def _scale(degp, h):
    return pl.pallas_call(
        _scale_body,
        grid=(N // BM,),
        in_specs=[pl.BlockSpec((NC, BM, DW), lambda i: (0, i, 0)),
                  pl.BlockSpec((BM, D), lambda i: (i, 0))],
        out_specs=[pl.BlockSpec((BM, 1), lambda i: (i, 0)),
                   pl.BlockSpec((BM, D), lambda i: (i, 0))],
        out_shape=[jax.ShapeDtypeStruct((N, 1), jnp.float32),
                   jax.ShapeDtypeStruct((N, D), jnp.float32)],
    )(degp, h)


def _mid_body(sp_ref, g_ref, dinv_ref, b_ref, w_ref, o_ref):
    ssum = sp_ref[0] + sp_ref[1]
    dinv = dinv_ref[...]
    x2 = jnp.maximum((ssum + g_ref[...]) * dinv + b_ref[...], 0.0)
    o_ref[...] = jnp.dot(x2, w_ref[...], preferred_element_type=jnp.float32) * dinv


def _mid(sp, g, dinv, b, w):
    return pl.pallas_call(
        _mid_body,
        grid=(N // BM,),
        in_specs=[pl.BlockSpec((NC, BM, D), lambda i: (0, i, 0)),
                  pl.BlockSpec((BM, D), lambda i: (i, 0)),
                  pl.BlockSpec((BM, 1), lambda i: (i, 0)),
                  pl.BlockSpec((1, D), lambda i: (0, 0)),
                  pl.BlockSpec((D, D), lambda i: (0, 0))],
        out_specs=pl.BlockSpec((BM, D), lambda i: (i, 0)),
        out_shape=jax.ShapeDtypeStruct((N, D), jnp.float32),
    )(sp, g, dinv, b, w)


def _fin_body(sp_ref, g_ref, dinv_ref, b_ref, o_ref):
    ssum = sp_ref[0] + sp_ref[1]
    o_ref[...] = jnp.maximum((ssum + g_ref[...]) * dinv_ref[...] + b_ref[...], 0.0)


def _fin(sp, g, dinv, b):
    return pl.pallas_call(
        _fin_body,
        grid=(N // BM,),
        in_specs=[pl.BlockSpec((NC, BM, D), lambda i: (0, i, 0)),
                  pl.BlockSpec((BM, D), lambda i: (i, 0)),
                  pl.BlockSpec((BM, 1), lambda i: (i, 0)),
                  pl.BlockSpec((1, D), lambda i: (0, 0))],
        out_specs=pl.BlockSpec((BM, D), lambda i: (i, 0)),
        out_shape=jax.ShapeDtypeStruct((N, D), jnp.float32),
    )(sp, g, dinv, b)


def kernel(x, edge_index, W1, b1, W2, b2):
    pad_s = jnp.zeros((EPAD - E,), jnp.int32)
    pad_d = jnp.full((EPAD - E,), N, jnp.int32)
    src = jnp.concatenate([edge_index[0], pad_s]).reshape(NW * NCHT, CHUNK)
    dst = jnp.concatenate([edge_index[1], pad_d]).reshape(NW * NCHT, CHUNK)
    degp = _deg_call(dst)
    h1 = _mm(x, W1)
    dinv, g1 = _scale(degp, h1)
    s1 = _scatter_call(g1, src, dst)
    g2 = _mid(s1, g1, dinv, b1.reshape(1, D), W2)
    s2 = _scatter_call(g2, src, dst)
    return _fin(s2, g2, dinv, b2.reshape(1, D))


# speedup vs baseline: 1.0020x; 1.0020x over previous
import jax
import jax.numpy as jnp
from jax import lax
from jax.experimental import pallas as pl
from jax.experimental.pallas import tpu as pltpu
from jax.experimental.pallas import tpu_sc as plsc

N = 10000
E = 320000
D = 128
NC = 2
NS = 16
NW = NC * NS
CHUNK = 128
NCHT = 80
EPAD = NW * NCHT * CHUNK
NROW = 10240
ROWS_PT = NROW // NS
IH = NCHT // 2
DW = 16

_MESH = plsc.VectorSubcoreMesh(core_axis_name="c", subcore_axis_name="s")
_SC_PARAMS = pltpu.CompilerParams(use_tc_tiling_on_sc=False)


def _deg_body(dst_hbm, out_hbm, idx_d, ones_v, zbuf, acc):
    c = lax.axis_index("c")
    s = lax.axis_index("s")
    wid = c * NS + s
    pltpu.sync_copy(dst_hbm.at[pl.ds(wid * NCHT, NCHT)], idx_d)

    def fill(i, carry):
        ones_v[i, :] = jnp.ones((DW,), jnp.float32)
        zbuf[i, :] = jnp.zeros((DW,), jnp.float32)
        return carry

    lax.fori_loop(0, ROWS_PT, fill, 0)
    pltpu.sync_copy(zbuf, acc.at[pl.ds(s * ROWS_PT, ROWS_PT)])
    plsc.subcore_barrier()

    def edge(j, carry):
        pltpu.sync_copy(ones_v.at[pl.ds(0, CHUNK)], acc.at[idx_d.at[j]], add=True)
        return carry

    lax.fori_loop(0, NCHT, edge, 0)
    plsc.subcore_barrier()
    pltpu.sync_copy(acc.at[pl.ds(s * ROWS_PT, ROWS_PT)],
                    out_hbm.at[c, pl.ds(s * ROWS_PT, ROWS_PT)])


_deg_call = pl.kernel(
    _deg_body,
    out_type=jax.ShapeDtypeStruct((NC, NROW, DW), jnp.float32),
    mesh=_MESH,
    scratch_types=[
        pltpu.VMEM((NCHT, CHUNK), jnp.int32),
        pltpu.VMEM((ROWS_PT, DW), jnp.float32),
        pltpu.VMEM((ROWS_PT, DW), jnp.float32),
        pltpu.VMEM_SHARED((NROW, DW), jnp.float32),
    ],
    compiler_params=_SC_PARAMS,
)


def _scatter_body(g_hbm, src_hbm, dst_hbm, out_hbm, idx_s, idx_d, rb0, rb1, acc,
                  sem0, sem1):
    c = lax.axis_index("c")
    s = lax.axis_index("s")
    wid = c * NS + s

    def zfill(i, carry):
        for j in range(D // 16):
            rb0[i, pl.ds(j * 16, 16)] = jnp.zeros((16,), jnp.float32)
        return carry

    lax.fori_loop(0, CHUNK, zfill, 0)

    def zcopy(k, carry):
        pltpu.sync_copy(rb0, acc.at[pl.ds(s * ROWS_PT + k * CHUNK, CHUNK)])
        return carry

    lax.fori_loop(0, ROWS_PT // CHUNK, zcopy, 0)
    plsc.subcore_barrier()

    for h in range(NCHT // IH):
        pltpu.sync_copy(src_hbm.at[pl.ds(wid * NCHT + h * IH, IH)], idx_s)
        pltpu.sync_copy(dst_hbm.at[pl.ds(wid * NCHT + h * IH, IH)], idx_d)
        pltpu.async_copy(g_hbm.at[idx_s.at[0]], rb0, sem0)

        def pair(i, carry):
            j = 2 * i
            pltpu.async_copy(g_hbm.at[idx_s.at[j + 1]], rb1, sem1)
            pltpu.make_async_copy(g_hbm.at[idx_s.at[j]], rb0, sem0).wait()
            pltpu.sync_copy(rb0, acc.at[idx_d.at[j]], add=True)

            @pl.when(j + 2 < IH)
            def _():
                pltpu.async_copy(g_hbm.at[idx_s.at[j + 2]], rb0, sem0)

            pltpu.make_async_copy(g_hbm.at[idx_s.at[j + 1]], rb1, sem1).wait()
            pltpu.sync_copy(rb1, acc.at[idx_d.at[j + 1]], add=True)
            return carry

        lax.fori_loop(0, IH // 2, pair, 0)
    plsc.subcore_barrier()
    pltpu.sync_copy(acc.at[pl.ds(s * ROWS_PT, ROWS_PT)],
                    out_hbm.at[c, pl.ds(s * ROWS_PT, ROWS_PT)])


_scatter_call = pl.kernel(
    _scatter_body,
    out_type=jax.ShapeDtypeStruct((NC, NROW, D), jnp.float32),
    mesh=_MESH,
    scratch_types=[
        pltpu.VMEM((IH, CHUNK), jnp.int32),
        pltpu.VMEM((IH, CHUNK), jnp.int32),
        pltpu.VMEM((CHUNK, D), jnp.float32),
        pltpu.VMEM((CHUNK, D), jnp.float32),
        pltpu.VMEM_SHARED((NROW, D), jnp.float32),
        pltpu.SemaphoreType.DMA,
        pltpu.SemaphoreType.DMA,
    ],
    compiler_params=_SC_PARAMS,
)

BM = 1000


def _mm_body(x_ref, w_ref, o_ref):
    o_ref[...] = jnp.dot(x_ref[...], w_ref[...], preferred_element_type=jnp.float32)


def _mm(x, w):
    return pl.pallas_call(
        _mm_body,
        grid=(N // BM,),
        in_specs=[pl.BlockSpec((BM, D), lambda i: (i, 0)),
                  pl.BlockSpec((D, D), lambda i: (0, 0))],
        out_specs=pl.BlockSpec((BM, D), lambda i: (i, 0)),
        out_shape=jax.ShapeDtypeStruct((N, D), jnp.float32),
    )(x, w)


def _scale_body(degp_ref, h_ref, dinv_ref, g_ref):
    deg = degp_ref[0, :, 0:1] + degp_ref[1, :, 0:1] + 1.0
    dinv = lax.rsqrt(deg)
    dinv_ref[...] = dinv
    g_ref[...] = h_ref[...] * dinv


def _scale(degp, h):
    return pl.pallas_call(
        _scale_body,
        grid=(N // BM,),
        in_specs=[pl.BlockSpec((NC, BM, DW), lambda i: (0, i, 0)),
                  pl.BlockSpec((BM, D), lambda i: (i, 0))],
        out_specs=[pl.BlockSpec((BM, 1), lambda i: (i, 0)),
                   pl.BlockSpec((BM, D), lambda i: (i, 0))],
        out_shape=[jax.ShapeDtypeStruct((N, 1), jnp.float32),
                   jax.ShapeDtypeStruct((N, D), jnp.float32)],
    )(degp, h)


def _mid_body(sp_ref, g_ref, dinv_ref, b_ref, w_ref, o_ref):
    ssum = sp_ref[0] + sp_ref[1]
    dinv = dinv_ref[...]
    x2 = jnp.maximum((ssum + g_ref[...]) * dinv + b_ref[...], 0.0)
    o_ref[...] = jnp.dot(x2, w_ref[...], preferred_element_type=jnp.float32) * dinv


def _mid(sp, g, dinv, b, w):
    return pl.pallas_call(
        _mid_body,
        grid=(N // BM,),
        in_specs=[pl.BlockSpec((NC, BM, D), lambda i: (0, i, 0)),
                  pl.BlockSpec((BM, D), lambda i: (i, 0)),
                  pl.BlockSpec((BM, 1), lambda i: (i, 0)),
                  pl.BlockSpec((1, D), lambda i: (0, 0)),
                  pl.BlockSpec((D, D), lambda i: (0, 0))],
        out_specs=pl.BlockSpec((BM, D), lambda i: (i, 0)),
        out_shape=jax.ShapeDtypeStruct((N, D), jnp.float32),
    )(sp, g, dinv, b, w)


def _fin_body(sp_ref, g_ref, dinv_ref, b_ref, o_ref):
    ssum = sp_ref[0] + sp_ref[1]
    o_ref[...] = jnp.maximum((ssum + g_ref[...]) * dinv_ref[...] + b_ref[...], 0.0)


def _fin(sp, g, dinv, b):
    return pl.pallas_call(
        _fin_body,
        grid=(N // BM,),
        in_specs=[pl.BlockSpec((NC, BM, D), lambda i: (0, i, 0)),
                  pl.BlockSpec((BM, D), lambda i: (i, 0)),
                  pl.BlockSpec((BM, 1), lambda i: (i, 0)),
                  pl.BlockSpec((1, D), lambda i: (0, 0))],
        out_specs=pl.BlockSpec((BM, D), lambda i: (i, 0)),
        out_shape=jax.ShapeDtypeStruct((N, D), jnp.float32),
    )(sp, g, dinv, b)


def kernel(x, edge_index, W1, b1, W2, b2):
    pad_s = jnp.zeros((EPAD - E,), jnp.int32)
    pad_d = N + (jnp.arange(EPAD - E, dtype=jnp.int32) % (NROW - N))
    src = jnp.concatenate([edge_index[0], pad_s]).reshape(NW * NCHT, CHUNK)
    dst = jnp.concatenate([edge_index[1], pad_d]).reshape(NW * NCHT, CHUNK)
    degp = _deg_call(dst)
    h1 = _mm(x, W1)
    dinv, g1 = _scale(degp, h1)
    s1 = _scatter_call(g1, src, dst)
    g2 = _mid(s1, g1, dinv, b1.reshape(1, D), W2)
    s2 = _scatter_call(g2, src, dst)
    return _fin(s2, g2, dinv, b2.reshape(1, D))


# speedup vs baseline: 2.9805x; 2.9745x over previous
import jax
import jax.numpy as jnp
from jax import lax
from jax.experimental import pallas as pl
from jax.experimental.pallas import tpu as pltpu
from jax.experimental.pallas import tpu_sc as plsc

N = 10000
E = 320000
D = 128
NC = 2
NS = 16
NW = NC * NS
CHUNK = 128
NCHT = 80
EPAD = NW * NCHT * CHUNK
NROW = 10240
ROWS_PT = NROW // NS
IH = NCHT // 2
DW = 16

_MESH = plsc.VectorSubcoreMesh(core_axis_name="c", subcore_axis_name="s")
_SC_PARAMS = pltpu.CompilerParams(use_tc_tiling_on_sc=False)


def _deg_body(dst_hbm, out_hbm, idx_d, ones_v, zbuf, acc):
    c = lax.axis_index("c")
    s = lax.axis_index("s")
    wid = c * NS + s
    pltpu.sync_copy(dst_hbm.at[pl.ds(wid * NCHT, NCHT)], idx_d)

    def fill(i, carry):
        ones_v[i, :] = jnp.ones((DW,), jnp.float32)
        zbuf[i, :] = jnp.zeros((DW,), jnp.float32)
        return carry

    lax.fori_loop(0, ROWS_PT, fill, 0)
    pltpu.sync_copy(zbuf, acc.at[pl.ds(s * ROWS_PT, ROWS_PT)])
    plsc.subcore_barrier()

    def edge(j, carry):
        pltpu.sync_copy(ones_v.at[pl.ds(0, CHUNK)], acc.at[idx_d.at[j]], add=True)
        return carry

    lax.fori_loop(0, NCHT, edge, 0)
    plsc.subcore_barrier()
    pltpu.sync_copy(acc.at[pl.ds(s * ROWS_PT, ROWS_PT)],
                    out_hbm.at[c, pl.ds(s * ROWS_PT, ROWS_PT)])


_deg_call = pl.kernel(
    _deg_body,
    out_type=jax.ShapeDtypeStruct((NC, NROW, DW), jnp.float32),
    mesh=_MESH,
    scratch_types=[
        pltpu.VMEM((NCHT, CHUNK), jnp.int32),
        pltpu.VMEM((ROWS_PT, DW), jnp.float32),
        pltpu.VMEM((ROWS_PT, DW), jnp.float32),
        pltpu.VMEM_SHARED((NROW, DW), jnp.float32),
    ],
    compiler_params=_SC_PARAMS,
)


def _scatter_body(g_hbm, src_hbm, dst_hbm, out_hbm, idx_s, idx_d, rb0, rb1, acc,
                  sem0, sem1):
    c = lax.axis_index("c")
    s = lax.axis_index("s")
    wid = c * NS + s

    def zfill(i, carry):
        for j in range(D // 16):
            rb0[i, pl.ds(j * 16, 16)] = jnp.zeros((16,), jnp.float32)
        return carry

    lax.fori_loop(0, CHUNK, zfill, 0)

    def zcopy(k, carry):
        pltpu.sync_copy(rb0, acc.at[pl.ds(s * ROWS_PT + k * CHUNK, CHUNK)])
        return carry

    lax.fori_loop(0, ROWS_PT // CHUNK, zcopy, 0)
    plsc.subcore_barrier()

    for h in range(NCHT // IH):
        pltpu.sync_copy(src_hbm.at[pl.ds(wid * NCHT + h * IH, IH)], idx_s)
        pltpu.sync_copy(dst_hbm.at[pl.ds(wid * NCHT + h * IH, IH)], idx_d)
        pltpu.async_copy(g_hbm.at[idx_s.at[0]], rb0, sem0)

        def pair(i, carry):
            j = 2 * i
            pltpu.async_copy(g_hbm.at[idx_s.at[j + 1]], rb1, sem1)
            pltpu.make_async_copy(g_hbm.at[idx_s.at[j]], rb0, sem0).wait()
            pltpu.sync_copy(rb0, acc.at[idx_d.at[j]], add=True)

            @pl.when(j + 2 < IH)
            def _():
                pltpu.async_copy(g_hbm.at[idx_s.at[j + 2]], rb0, sem0)

            pltpu.make_async_copy(g_hbm.at[idx_s.at[j + 1]], rb1, sem1).wait()
            pltpu.sync_copy(rb1, acc.at[idx_d.at[j + 1]], add=True)
            return carry

        lax.fori_loop(0, IH // 2, pair, 0)
    plsc.subcore_barrier()
    pltpu.sync_copy(acc.at[pl.ds(s * ROWS_PT, ROWS_PT)],
                    out_hbm.at[c, pl.ds(s * ROWS_PT, ROWS_PT)])


_scatter_call = pl.kernel(
    _scatter_body,
    out_type=jax.ShapeDtypeStruct((NC, NROW, D), jnp.float32),
    mesh=_MESH,
    scratch_types=[
        pltpu.VMEM((IH, CHUNK), jnp.int32),
        pltpu.VMEM((IH, CHUNK), jnp.int32),
        pltpu.VMEM((CHUNK, D), jnp.float32),
        pltpu.VMEM((CHUNK, D), jnp.float32),
        pltpu.VMEM_SHARED((NROW, D), jnp.float32),
        pltpu.SemaphoreType.DMA,
        pltpu.SemaphoreType.DMA,
    ],
    compiler_params=_SC_PARAMS,
)

BM = 1000


def _mm_body(x_ref, w_ref, o_ref):
    o_ref[...] = jnp.dot(x_ref[...], w_ref[...], preferred_element_type=jnp.float32)


def _mm(x, w):
    return pl.pallas_call(
        _mm_body,
        grid=(N // BM,),
        in_specs=[pl.BlockSpec((BM, D), lambda i: (i, 0)),
                  pl.BlockSpec((D, D), lambda i: (0, 0))],
        out_specs=pl.BlockSpec((BM, D), lambda i: (i, 0)),
        out_shape=jax.ShapeDtypeStruct((N, D), jnp.float32),
    )(x, w)


def _scale_body(degp_ref, h_ref, dinv_ref, g_ref):
    deg = degp_ref[0, :, 0:1] + degp_ref[1, :, 0:1] + 1.0
    dinv = lax.rsqrt(deg)
    dinv_ref[...] = dinv
    g_ref[...] = h_ref[...] * dinv


def _scale(degp, h):
    return pl.pallas_call(
        _scale_body,
        grid=(N // BM,),
        in_specs=[pl.BlockSpec((NC, BM, DW), lambda i: (0, i, 0)),
                  pl.BlockSpec((BM, D), lambda i: (i, 0))],
        out_specs=[pl.BlockSpec((BM, 1), lambda i: (i, 0)),
                   pl.BlockSpec((BM, D), lambda i: (i, 0))],
        out_shape=[jax.ShapeDtypeStruct((N, 1), jnp.float32),
                   jax.ShapeDtypeStruct((N, D), jnp.float32)],
    )(degp, h)


def _mid_body(sp_ref, g_ref, dinv_ref, b_ref, w_ref, o_ref):
    ssum = sp_ref[0] + sp_ref[1]
    dinv = dinv_ref[...]
    x2 = jnp.maximum((ssum + g_ref[...]) * dinv + b_ref[...], 0.0)
    o_ref[...] = jnp.dot(x2, w_ref[...], preferred_element_type=jnp.float32) * dinv


def _mid(sp, g, dinv, b, w):
    return pl.pallas_call(
        _mid_body,
        grid=(N // BM,),
        in_specs=[pl.BlockSpec((NC, BM, D), lambda i: (0, i, 0)),
                  pl.BlockSpec((BM, D), lambda i: (i, 0)),
                  pl.BlockSpec((BM, 1), lambda i: (i, 0)),
                  pl.BlockSpec((1, D), lambda i: (0, 0)),
                  pl.BlockSpec((D, D), lambda i: (0, 0))],
        out_specs=pl.BlockSpec((BM, D), lambda i: (i, 0)),
        out_shape=jax.ShapeDtypeStruct((N, D), jnp.float32),
    )(sp, g, dinv, b, w)


def _fin_body(sp_ref, g_ref, dinv_ref, b_ref, o_ref):
    ssum = sp_ref[0] + sp_ref[1]
    o_ref[...] = jnp.maximum((ssum + g_ref[...]) * dinv_ref[...] + b_ref[...], 0.0)


def _fin(sp, g, dinv, b):
    return pl.pallas_call(
        _fin_body,
        grid=(N // BM,),
        in_specs=[pl.BlockSpec((NC, BM, D), lambda i: (0, i, 0)),
                  pl.BlockSpec((BM, D), lambda i: (i, 0)),
                  pl.BlockSpec((BM, 1), lambda i: (i, 0)),
                  pl.BlockSpec((1, D), lambda i: (0, 0))],
        out_specs=pl.BlockSpec((BM, D), lambda i: (i, 0)),
        out_shape=jax.ShapeDtypeStruct((N, D), jnp.float32),
    )(sp, g, dinv, b)


def kernel(x, edge_index, W1, b1, W2, b2):
    pad_s = jnp.arange(EPAD - E, dtype=jnp.int32) % N
    pad_d = N + (jnp.arange(EPAD - E, dtype=jnp.int32) % (NROW - N))
    src = jnp.concatenate([edge_index[0], pad_s]).reshape(NW * NCHT, CHUNK)
    dst = jnp.concatenate([edge_index[1], pad_d]).reshape(NW * NCHT, CHUNK)
    degp = _deg_call(dst)
    h1 = _mm(x, W1)
    dinv, g1 = _scale(degp, h1)
    s1 = _scatter_call(g1, src, dst)
    g2 = _mid(s1, g1, dinv, b1.reshape(1, D), W2)
    s2 = _scatter_call(g2, src, dst)
    return _fin(s2, g2, dinv, b2.reshape(1, D))


# speedup vs baseline: 3.0619x; 1.0273x over previous
import jax
import jax.numpy as jnp
from jax import lax
from jax.experimental import pallas as pl
from jax.experimental.pallas import tpu as pltpu
from jax.experimental.pallas import tpu_sc as plsc

N = 10000
E = 320000
D = 128
NC = 2
NS = 16
NW = NC * NS
CHUNK = 64
NCHT = 160
EPAD = NW * NCHT * CHUNK
NROW = 10240
ROWS_PT = NROW // NS
IH = NCHT // 2
DW = 16

_MESH = plsc.VectorSubcoreMesh(core_axis_name="c", subcore_axis_name="s")
_SC_PARAMS = pltpu.CompilerParams(use_tc_tiling_on_sc=False)


def _deg_body(dst_hbm, out_hbm, idx_d, ones_v, zbuf, acc):
    c = lax.axis_index("c")
    s = lax.axis_index("s")
    wid = c * NS + s
    pltpu.sync_copy(dst_hbm.at[pl.ds(wid * NCHT, NCHT)], idx_d)

    def fill(i, carry):
        ones_v[i, :] = jnp.ones((DW,), jnp.float32)
        zbuf[i, :] = jnp.zeros((DW,), jnp.float32)
        return carry

    lax.fori_loop(0, ROWS_PT, fill, 0)
    pltpu.sync_copy(zbuf, acc.at[pl.ds(s * ROWS_PT, ROWS_PT)])
    plsc.subcore_barrier()

    def edge(j, carry):
        pltpu.sync_copy(ones_v.at[pl.ds(0, CHUNK)], acc.at[idx_d.at[j]], add=True)
        return carry

    lax.fori_loop(0, NCHT, edge, 0)
    plsc.subcore_barrier()
    pltpu.sync_copy(acc.at[pl.ds(s * ROWS_PT, ROWS_PT)],
                    out_hbm.at[c, pl.ds(s * ROWS_PT, ROWS_PT)])


_deg_call = pl.kernel(
    _deg_body,
    out_type=jax.ShapeDtypeStruct((NC, NROW, DW), jnp.float32),
    mesh=_MESH,
    scratch_types=[
        pltpu.VMEM((NCHT, CHUNK), jnp.int32),
        pltpu.VMEM((ROWS_PT, DW), jnp.float32),
        pltpu.VMEM((ROWS_PT, DW), jnp.float32),
        pltpu.VMEM_SHARED((NROW, DW), jnp.float32),
    ],
    compiler_params=_SC_PARAMS,
)


def _scatter_body(g_hbm, src_hbm, dst_hbm, out_hbm, idx_s, idx_d,
                  rb0, rb1, rb2, rb3, acc, sem0, sem1, sem2, sem3):
    c = lax.axis_index("c")
    s = lax.axis_index("s")
    wid = c * NS + s
    rbs = (rb0, rb1, rb2, rb3)
    sems = (sem0, sem1, sem2, sem3)

    def zfill(i, carry):
        for j in range(D // 16):
            rb0[i, pl.ds(j * 16, 16)] = jnp.zeros((16,), jnp.float32)
        return carry

    lax.fori_loop(0, CHUNK, zfill, 0)

    def zcopy(k, carry):
        pltpu.sync_copy(rb0, acc.at[pl.ds(s * ROWS_PT + k * CHUNK, CHUNK)])
        return carry

    lax.fori_loop(0, ROWS_PT // CHUNK, zcopy, 0)
    plsc.subcore_barrier()

    for h in range(NCHT // IH):
        pltpu.sync_copy(src_hbm.at[pl.ds(wid * NCHT + h * IH, IH)], idx_s)
        pltpu.sync_copy(dst_hbm.at[pl.ds(wid * NCHT + h * IH, IH)], idx_d)
        for b in range(3):
            pltpu.async_copy(g_hbm.at[idx_s.at[b]], rbs[b], sems[b])

        def quad(i, carry):
            j = 4 * i
            for b in range(4):
                jj = j + b
                nb = (b + 3) % 4
                pltpu.make_async_copy(g_hbm.at[idx_s.at[jj]], rbs[b], sems[b]).wait()
                pltpu.sync_copy(rbs[b], acc.at[idx_d.at[jj]], add=True)

                @pl.when(jj + 3 < IH)
                def _(jj=jj, nb=nb):
                    pltpu.async_copy(g_hbm.at[idx_s.at[jj + 3]], rbs[nb], sems[nb])
            return carry

        lax.fori_loop(0, IH // 4, quad, 0)
    plsc.subcore_barrier()
    pltpu.sync_copy(acc.at[pl.ds(s * ROWS_PT, ROWS_PT)],
                    out_hbm.at[c, pl.ds(s * ROWS_PT, ROWS_PT)])


_scatter_call = pl.kernel(
    _scatter_body,
    out_type=jax.ShapeDtypeStruct((NC, NROW, D), jnp.float32),
    mesh=_MESH,
    scratch_types=[
        pltpu.VMEM((IH, CHUNK), jnp.int32),
        pltpu.VMEM((IH, CHUNK), jnp.int32),
        pltpu.VMEM((CHUNK, D), jnp.float32),
        pltpu.VMEM((CHUNK, D), jnp.float32),
        pltpu.VMEM((CHUNK, D), jnp.float32),
        pltpu.VMEM((CHUNK, D), jnp.float32),
        pltpu.VMEM_SHARED((NROW, D), jnp.float32),
        pltpu.SemaphoreType.DMA,
        pltpu.SemaphoreType.DMA,
        pltpu.SemaphoreType.DMA,
        pltpu.SemaphoreType.DMA,
    ],
    compiler_params=_SC_PARAMS,
)

BM = 1000


def _mm_body(x_ref, w_ref, o_ref):
    o_ref[...] = jnp.dot(x_ref[...], w_ref[...], preferred_element_type=jnp.float32)


def _mm(x, w):
    return pl.pallas_call(
        _mm_body,
        grid=(N // BM,),
        in_specs=[pl.BlockSpec((BM, D), lambda i: (i, 0)),
                  pl.BlockSpec((D, D), lambda i: (0, 0))],
        out_specs=pl.BlockSpec((BM, D), lambda i: (i, 0)),
        out_shape=jax.ShapeDtypeStruct((N, D), jnp.float32),
    )(x, w)


def _scale_body(degp_ref, h_ref, dinv_ref, g_ref):
    deg = degp_ref[0, :, 0:1] + degp_ref[1, :, 0:1] + 1.0
    dinv = lax.rsqrt(deg)
    dinv_ref[...] = dinv
    g_ref[...] = h_ref[...] * dinv


def _scale(degp, h):
    return pl.pallas_call(
        _scale_body,
        grid=(N // BM,),
        in_specs=[pl.BlockSpec((NC, BM, DW), lambda i: (0, i, 0)),
                  pl.BlockSpec((BM, D), lambda i: (i, 0))],
        out_specs=[pl.BlockSpec((BM, 1), lambda i: (i, 0)),
                   pl.BlockSpec((BM, D), lambda i: (i, 0))],
        out_shape=[jax.ShapeDtypeStruct((N, 1), jnp.float32),
                   jax.ShapeDtypeStruct((N, D), jnp.float32)],
    )(degp, h)


def _mid_body(sp_ref, g_ref, dinv_ref, b_ref, w_ref, o_ref):
    ssum = sp_ref[0] + sp_ref[1]
    dinv = dinv_ref[...]
    x2 = jnp.maximum((ssum + g_ref[...]) * dinv + b_ref[...], 0.0)
    o_ref[...] = jnp.dot(x2, w_ref[...], preferred_element_type=jnp.float32) * dinv


def _mid(sp, g, dinv, b, w):
    return pl.pallas_call(
        _mid_body,
        grid=(N // BM,),
        in_specs=[pl.BlockSpec((NC, BM, D), lambda i: (0, i, 0)),
                  pl.BlockSpec((BM, D), lambda i: (i, 0)),
                  pl.BlockSpec((BM, 1), lambda i: (i, 0)),
                  pl.BlockSpec((1, D), lambda i: (0, 0)),
                  pl.BlockSpec((D, D), lambda i: (0, 0))],
        out_specs=pl.BlockSpec((BM, D), lambda i: (i, 0)),
        out_shape=jax.ShapeDtypeStruct((N, D), jnp.float32),
    )(sp, g, dinv, b, w)


def _fin_body(sp_ref, g_ref, dinv_ref, b_ref, o_ref):
    ssum = sp_ref[0] + sp_ref[1]
    o_ref[...] = jnp.maximum((ssum + g_ref[...]) * dinv_ref[...] + b_ref[...], 0.0)


def _fin(sp, g, dinv, b):
    return pl.pallas_call(
        _fin_body,
        grid=(N // BM,),
        in_specs=[pl.BlockSpec((NC, BM, D), lambda i: (0, i, 0)),
                  pl.BlockSpec((BM, D), lambda i: (i, 0)),
                  pl.BlockSpec((BM, 1), lambda i: (i, 0)),
                  pl.BlockSpec((1, D), lambda i: (0, 0))],
        out_specs=pl.BlockSpec((BM, D), lambda i: (i, 0)),
        out_shape=jax.ShapeDtypeStruct((N, D), jnp.float32),
    )(sp, g, dinv, b)


def kernel(x, edge_index, W1, b1, W2, b2):
    pad_s = jnp.arange(EPAD - E, dtype=jnp.int32) % N
    pad_d = N + (jnp.arange(EPAD - E, dtype=jnp.int32) % (NROW - N))
    src = jnp.concatenate([edge_index[0], pad_s]).reshape(NW * NCHT, CHUNK)
    dst = jnp.concatenate([edge_index[1], pad_d]).reshape(NW * NCHT, CHUNK)
    degp = _deg_call(dst)
    h1 = _mm(x, W1)
    dinv, g1 = _scale(degp, h1)
    s1 = _scatter_call(g1, src, dst)
    g2 = _mid(s1, g1, dinv, b1.reshape(1, D), W2)
    s2 = _scatter_call(g2, src, dst)
    return _fin(s2, g2, dinv, b2.reshape(1, D))


# speedup vs baseline: 3.1093x; 1.0155x over previous
import jax
import jax.numpy as jnp
from jax import lax
from jax.experimental import pallas as pl
from jax.experimental.pallas import tpu as pltpu
from jax.experimental.pallas import tpu_sc as plsc

N = 10000
E = 320000
D = 128
NC = 2
NS = 16
NW = NC * NS
CHUNK = 64
NCHT = 160
EPAD = NW * NCHT * CHUNK
NROW = 10240
ROWS_PT = NROW // NS
IH = NCHT // 2
DW = 16

_MESH = plsc.VectorSubcoreMesh(core_axis_name="c", subcore_axis_name="s")
_SC_PARAMS = pltpu.CompilerParams(use_tc_tiling_on_sc=False)


def _deg_body(dst_hbm, out_hbm, idx_d, ones_v, zbuf, acc):
    c = lax.axis_index("c")
    s = lax.axis_index("s")
    wid = c * NS + s
    pltpu.sync_copy(dst_hbm.at[pl.ds(wid * NCHT, NCHT)], idx_d)

    def fill(i, carry):
        ones_v[i, :] = jnp.ones((DW,), jnp.float32)
        zbuf[i, :] = jnp.zeros((DW,), jnp.float32)
        return carry

    lax.fori_loop(0, ROWS_PT, fill, 0)
    pltpu.sync_copy(zbuf, acc.at[pl.ds(s * ROWS_PT, ROWS_PT)])
    plsc.subcore_barrier()

    def edge(j, carry):
        pltpu.sync_copy(ones_v.at[pl.ds(0, CHUNK)], acc.at[idx_d.at[j]], add=True)
        return carry

    lax.fori_loop(0, NCHT, edge, 0)
    plsc.subcore_barrier()
    pltpu.sync_copy(acc.at[pl.ds(s * ROWS_PT, ROWS_PT)],
                    out_hbm.at[pl.ds(s * ROWS_PT, ROWS_PT), pl.ds(c * DW, DW)])


_deg_call = pl.kernel(
    _deg_body,
    out_type=jax.ShapeDtypeStruct((NROW, D), jnp.float32),
    mesh=_MESH,
    scratch_types=[
        pltpu.VMEM((NCHT, CHUNK), jnp.int32),
        pltpu.VMEM((ROWS_PT, DW), jnp.float32),
        pltpu.VMEM((ROWS_PT, DW), jnp.float32),
        pltpu.VMEM_SHARED((NROW, DW), jnp.float32),
    ],
    compiler_params=_SC_PARAMS,
)


def _scatter_body(g_hbm, src_hbm, dst_hbm, out_hbm, idx_s, idx_d,
                  rb0, rb1, rb2, rb3, acc, sem0, sem1, sem2, sem3):
    c = lax.axis_index("c")
    s = lax.axis_index("s")
    wid = c * NS + s
    rbs = (rb0, rb1, rb2, rb3)
    sems = (sem0, sem1, sem2, sem3)

    def zfill(i, carry):
        for j in range(D // 16):
            rb0[i, pl.ds(j * 16, 16)] = jnp.zeros((16,), jnp.float32)
        return carry

    lax.fori_loop(0, CHUNK, zfill, 0)

    def zcopy(k, carry):
        pltpu.sync_copy(rb0, acc.at[pl.ds(s * ROWS_PT + k * CHUNK, CHUNK)])
        return carry

    lax.fori_loop(0, ROWS_PT // CHUNK, zcopy, 0)
    plsc.subcore_barrier()

    for h in range(NCHT // IH):
        pltpu.sync_copy(src_hbm.at[pl.ds(wid * NCHT + h * IH, IH)], idx_s)
        pltpu.sync_copy(dst_hbm.at[pl.ds(wid * NCHT + h * IH, IH)], idx_d)
        for b in range(3):
            pltpu.async_copy(g_hbm.at[idx_s.at[b]], rbs[b], sems[b])

        def quad(i, carry):
            j = 4 * i
            for b in range(4):
                jj = j + b
                nb = (b + 3) % 4
                pltpu.make_async_copy(g_hbm.at[idx_s.at[jj]], rbs[b], sems[b]).wait()
                pltpu.sync_copy(rbs[b], acc.at[idx_d.at[jj]], add=True)

                @pl.when(jj + 3 < IH)
                def _(jj=jj, nb=nb):
                    pltpu.async_copy(g_hbm.at[idx_s.at[jj + 3]], rbs[nb], sems[nb])
            return carry

        lax.fori_loop(0, IH // 4, quad, 0)
    plsc.subcore_barrier()
    pltpu.sync_copy(acc.at[pl.ds(s * ROWS_PT, ROWS_PT)],
                    out_hbm.at[c, pl.ds(s * ROWS_PT, ROWS_PT)])


_scatter_call = pl.kernel(
    _scatter_body,
    out_type=jax.ShapeDtypeStruct((NC, NROW, D), jnp.float32),
    mesh=_MESH,
    scratch_types=[
        pltpu.VMEM((IH, CHUNK), jnp.int32),
        pltpu.VMEM((IH, CHUNK), jnp.int32),
        pltpu.VMEM((CHUNK, D), jnp.float32),
        pltpu.VMEM((CHUNK, D), jnp.float32),
        pltpu.VMEM((CHUNK, D), jnp.float32),
        pltpu.VMEM((CHUNK, D), jnp.float32),
        pltpu.VMEM_SHARED((NROW, D), jnp.float32),
        pltpu.SemaphoreType.DMA,
        pltpu.SemaphoreType.DMA,
        pltpu.SemaphoreType.DMA,
        pltpu.SemaphoreType.DMA,
    ],
    compiler_params=_SC_PARAMS,
)

BM = 1000


def _mm_body(x_ref, w_ref, o_ref):
    o_ref[...] = jnp.dot(x_ref[...], w_ref[...], preferred_element_type=jnp.float32)


def _mm(x, w):
    return pl.pallas_call(
        _mm_body,
        grid=(N // BM,),
        in_specs=[pl.BlockSpec((BM, D), lambda i: (i, 0)),
                  pl.BlockSpec((D, D), lambda i: (0, 0))],
        out_specs=pl.BlockSpec((BM, D), lambda i: (i, 0)),
        out_shape=jax.ShapeDtypeStruct((N, D), jnp.float32),
    )(x, w)


def _scale_body(degp_ref, h_ref, dinv_ref, g_ref):
    deg = degp_ref[:, 0:1] + degp_ref[:, DW:DW + 1] + 1.0
    dinv = lax.rsqrt(deg)
    dinv_ref[...] = dinv
    g_ref[...] = h_ref[...] * dinv


def _scale(degp, h):
    return pl.pallas_call(
        _scale_body,
        grid=(N // BM,),
        in_specs=[pl.BlockSpec((BM, D), lambda i: (i, 0)),
                  pl.BlockSpec((BM, D), lambda i: (i, 0))],
        out_specs=[pl.BlockSpec((BM, 1), lambda i: (i, 0)),
                   pl.BlockSpec((BM, D), lambda i: (i, 0))],
        out_shape=[jax.ShapeDtypeStruct((N, 1), jnp.float32),
                   jax.ShapeDtypeStruct((N, D), jnp.float32)],
    )(degp, h)


def _mid_body(sp_ref, g_ref, dinv_ref, b_ref, w_ref, o_ref):
    ssum = sp_ref[0] + sp_ref[1]
    dinv = dinv_ref[...]
    x2 = jnp.maximum((ssum + g_ref[...]) * dinv + b_ref[...], 0.0)
    o_ref[...] = jnp.dot(x2, w_ref[...], preferred_element_type=jnp.float32) * dinv


def _mid(sp, g, dinv, b, w):
    return pl.pallas_call(
        _mid_body,
        grid=(N // BM,),
        in_specs=[pl.BlockSpec((NC, BM, D), lambda i: (0, i, 0)),
                  pl.BlockSpec((BM, D), lambda i: (i, 0)),
                  pl.BlockSpec((BM, 1), lambda i: (i, 0)),
                  pl.BlockSpec((1, D), lambda i: (0, 0)),
                  pl.BlockSpec((D, D), lambda i: (0, 0))],
        out_specs=pl.BlockSpec((BM, D), lambda i: (i, 0)),
        out_shape=jax.ShapeDtypeStruct((N, D), jnp.float32),
    )(sp, g, dinv, b, w)


def _fin_body(sp_ref, g_ref, dinv_ref, b_ref, o_ref):
    ssum = sp_ref[0] + sp_ref[1]
    o_ref[...] = jnp.maximum((ssum + g_ref[...]) * dinv_ref[...] + b_ref[...], 0.0)


def _fin(sp, g, dinv, b):
    return pl.pallas_call(
        _fin_body,
        grid=(N // BM,),
        in_specs=[pl.BlockSpec((NC, BM, D), lambda i: (0, i, 0)),
                  pl.BlockSpec((BM, D), lambda i: (i, 0)),
                  pl.BlockSpec((BM, 1), lambda i: (i, 0)),
                  pl.BlockSpec((1, D), lambda i: (0, 0))],
        out_specs=pl.BlockSpec((BM, D), lambda i: (i, 0)),
        out_shape=jax.ShapeDtypeStruct((N, D), jnp.float32),
    )(sp, g, dinv, b)


def kernel(x, edge_index, W1, b1, W2, b2):
    pad_s = jnp.arange(EPAD - E, dtype=jnp.int32) % N
    pad_d = N + (jnp.arange(EPAD - E, dtype=jnp.int32) % (NROW - N))
    src = jnp.concatenate([edge_index[0], pad_s]).reshape(NW * NCHT, CHUNK)
    dst = jnp.concatenate([edge_index[1], pad_d]).reshape(NW * NCHT, CHUNK)
    degp = _deg_call(dst)
    h1 = _mm(x, W1)
    dinv, g1 = _scale(degp, h1)
    s1 = _scatter_call(g1, src, dst)
    g2 = _mid(s1, g1, dinv, b1.reshape(1, D), W2)
    s2 = _scatter_call(g2, src, dst)
    return _fin(s2, g2, dinv, b2.reshape(1, D))
